# Initial kernel scaffold; baseline (speedup 1.0000x reference)
import jax
import jax.numpy as jnp
from jax.experimental import pallas as pl


def kernel(embeds, edge_weight, edge_index, users, pos_items, neg_items):
    raise NotImplementedError("write your pallas kernel here")


# speedup vs baseline: 2.9545x; 2.9545x over previous
import functools

import jax
import jax.numpy as jnp
from jax import lax
from jax.experimental import pallas as pl
from jax.experimental.pallas import tpu as pltpu
from jax.experimental.pallas import tpu_sc as plsc

N_USERS = 5000
N_NODES = 10000
D = 128
E = 320000
B = 4096
N_HOPS = 3

NC = 2
NS = 16
L = 16
NW = NC * NS
CHUNK = 128
NCHUNKS = 80
EPW = NCHUNKS * CHUNK
E_PAD = NW * EPW
N_PAD = 10240
ROWS_PER_TILE = N_PAD // NS
ZROWS = 128
BPW = B // NW
NG = D // L

_mesh = plsc.VectorSubcoreMesh(
    core_axis_name="c", subcore_axis_name="s", num_cores=NC, num_subcores=NS
)


def _make_hop():
    def body(tbl, ridx3, cidx3, w3, out, acc, ridx_v, cidx_v, w_v, rows_v, sem):
        cid = lax.axis_index("c")
        sid = lax.axis_index("s")
        wid = sid * NC + cid

        pltpu.sync_copy(ridx3.at[wid], ridx_v)
        pltpu.sync_copy(cidx3.at[wid], cidx_v)
        pltpu.sync_copy(w3.at[wid], w_v)

        def _zrow(r, _):
            for g in range(NG):
                rows_v[r, pl.ds(g * L, L)] = jnp.zeros((L,), jnp.float32)
            return 0

        lax.fori_loop(0, ZROWS, _zrow, 0)
        for k in range(ROWS_PER_TILE // ZROWS):
            pltpu.sync_copy(
                rows_v, acc.at[pl.ds(sid * ROWS_PER_TILE + k * ZROWS, ZROWS)]
            )
        plsc.subcore_barrier()

        def chunk(j, _):
            pltpu.async_copy(tbl.at[ridx_v.at[j]], rows_v, sem).wait()

            def scale(eb, _):
                w16 = w_v[j, pl.ds(eb * L, L)]
                for e16 in range(L):
                    e = eb * L + e16
                    w = w16[e16]
                    for g in range(NG):
                        rows_v[e, pl.ds(g * L, L)] = rows_v[e, pl.ds(g * L, L)] * w
                return 0

            lax.fori_loop(0, CHUNK // L, scale, 0)
            pltpu.sync_copy(rows_v, acc.at[cidx_v.at[j]], add=True)
            return 0

        lax.fori_loop(0, NCHUNKS, chunk, 0)
        plsc.subcore_barrier()

        for k in range(ROWS_PER_TILE // ZROWS):
            off = sid * ROWS_PER_TILE + k * ZROWS
            pltpu.sync_copy(acc.at[pl.ds(off, ZROWS)], rows_v)
            pltpu.sync_copy(rows_v, out.at[cid, pl.ds(off, ZROWS)])

    return pl.kernel(
        body,
        out_type=jax.ShapeDtypeStruct((NC, N_PAD, D), jnp.float32),
        mesh=_mesh,
        scratch_types=[
            pltpu.VMEM_SHARED((N_PAD, D), jnp.float32),
            pltpu.VMEM((NCHUNKS, CHUNK), jnp.int32),
            pltpu.VMEM((NCHUNKS, CHUNK), jnp.int32),
            pltpu.VMEM((NCHUNKS, CHUNK), jnp.float32),
            pltpu.VMEM((CHUNK, D), jnp.float32),
            pltpu.SemaphoreType.DMA,
        ],
    )


def _make_tail():
    def body(emb, t1, t2, p3, users2, pos2, neg2,
             out_u, out_p, out_n, out_ru, out_rp, out_rn,
             idx_v, raw_v, acc_v, tmp_v, sem):
        cid = lax.axis_index("c")
        sid = lax.axis_index("s")
        wid = sid * NC + cid

        for idx_hbm, out_pool, out_raw in (
            (users2, out_u, out_ru),
            (pos2, out_p, out_rp),
            (neg2, out_n, out_rn),
        ):
            pltpu.sync_copy(idx_hbm.at[wid], idx_v)
            pltpu.async_copy(emb.at[idx_v], raw_v, sem).wait()
            pltpu.sync_copy(raw_v, out_raw.at[pl.ds(wid * BPW, BPW)])
            pltpu.async_copy(emb.at[idx_v], acc_v, sem).wait()
            for tbl in (t1, t2, p3.at[0], p3.at[1]):
                pltpu.async_copy(tbl.at[idx_v], tmp_v, sem).wait()

                def add(e, _):
                    for g in range(NG):
                        s = (e, pl.ds(g * L, L))
                        acc_v[s] = acc_v[s] + tmp_v[s]
                    return 0

                lax.fori_loop(0, BPW, add, 0)

            quarter = jnp.float32(0.25)

            def scl(e, _):
                for g in range(NG):
                    s = (e, pl.ds(g * L, L))
                    acc_v[s] = acc_v[s] * quarter
                return 0

            lax.fori_loop(0, BPW, scl, 0)
            pltpu.sync_copy(acc_v, out_pool.at[pl.ds(wid * BPW, BPW)])

    shp = jax.ShapeDtypeStruct((B, D), jnp.float32)
    return pl.kernel(
        body,
        out_type=(shp, shp, shp, shp, shp, shp),
        mesh=_mesh,
        scratch_types=[
            pltpu.VMEM((BPW,), jnp.int32),
            pltpu.VMEM((BPW, D), jnp.float32),
            pltpu.VMEM((BPW, D), jnp.float32),
            pltpu.VMEM((BPW, D), jnp.float32),
            pltpu.SemaphoreType.DMA,
        ],
    )


_hop = _make_hop()
_tail = _make_tail()


def kernel(embeds, edge_weight, edge_index, users, pos_items, neg_items):
    pad_idx = jnp.zeros((2, E_PAD - E), jnp.int32)
    pad_w = jnp.zeros((E_PAD - E,), jnp.float32)
    eidx = jnp.concatenate([edge_index, pad_idx], axis=1)
    ew = jnp.concatenate([edge_weight, pad_w], axis=0)
    ridx3 = eidx[0].reshape(NW, NCHUNKS, CHUNK)
    cidx3 = eidx[1].reshape(NW, NCHUNKS, CHUNK)
    w3 = ew.reshape(NW, NCHUNKS, CHUNK)

    emb_pad = jnp.concatenate(
        [embeds, jnp.zeros((N_PAD - N_NODES, D), jnp.float32)], axis=0
    )
    p1 = _hop(emb_pad, ridx3, cidx3, w3)
    t1 = p1[0] + p1[1]
    p2 = _hop(t1, ridx3, cidx3, w3)
    t2 = p2[0] + p2[1]
    p3 = _hop(t2, ridx3, cidx3, w3)

    u2 = users.reshape(NW, BPW)
    pp2 = pos_items.reshape(NW, BPW)
    nn2 = neg_items.reshape(NW, BPW)
    return _tail(embeds, t1, t2, p3, u2, pp2, nn2)
